# baseline (speedup 1.0000x reference)
import functools

import jax
import jax.numpy as jnp
from jax import lax
from jax.experimental import pallas as pl
from jax.experimental.pallas import tpu as pltpu
from jax.experimental.pallas import tpu_sc as plsc

DIM = 16
BATCH = 16384

_NC = 2
_NS = 16
_NW = _NC * _NS
_BPW = BATCH // _NW
_CHUNK = 128
_NCHUNK = _BPW // _CHUNK
_GROUPS = _BPW // 16


def _newton_rsqrt(p):
    i = plsc.bitcast(p, jnp.int32)
    i = jnp.int32(0x5F3759DF) - (i >> 1)
    y = plsc.bitcast(i, jnp.float32)
    for _ in range(3):
        y = y * (jnp.float32(1.5) - jnp.float32(0.5) * p * y * y)
    return y


def _make_kernel():
    mesh = plsc.VectorSubcoreMesh(core_axis_name="c", subcore_axis_name="s")

    @functools.partial(
        pl.kernel,
        mesh=mesh,
        out_type=jax.ShapeDtypeStruct((BATCH,), jnp.float32),
        compiler_params=pltpu.CompilerParams(
            needs_layout_passes=False, use_tc_tiling_on_sc=False
        ),
        scratch_types=[
            pltpu.VMEM((_NCHUNK, _CHUNK), jnp.int32),
            pltpu.VMEM((_BPW, DIM), jnp.float32),
            pltpu.VMEM((_BPW, DIM), jnp.float32),
            pltpu.VMEM((_BPW,), jnp.float32),
            pltpu.SemaphoreType.DMA,
        ],
    )
    def k(e1_hbm, e2_hbm, idx_hbm, out_hbm,
          idx_v, a_v, b_v, out_v, sem):
        wid = lax.axis_index("s") * _NC + lax.axis_index("c")

        pltpu.sync_copy(idx_hbm.at[pl.ds(wid * _NCHUNK, _NCHUNK)], idx_v)

        copies = []
        for j in range(_NCHUNK):
            dst = pl.ds(j * _CHUNK, _CHUNK)
            copies.append(pltpu.async_copy(e1_hbm.at[idx_v.at[j]], a_v.at[dst], sem))
            copies.append(pltpu.async_copy(e2_hbm.at[idx_v.at[j]], b_v.at[dst], sem))
        for cp in copies:
            cp.wait()

        lane = lax.iota(jnp.int32, 16)

        def body(g, carry):
            rows = g * 16 + lane
            num = jnp.zeros((16,), jnp.float32)
            na = jnp.zeros((16,), jnp.float32)
            nb = jnp.zeros((16,), jnp.float32)
            for d in range(DIM):
                col = jnp.full((16,), d, jnp.int32)
                av = plsc.load_gather(a_v, [rows, col])
                bv = plsc.load_gather(b_v, [rows, col])
                num = num + av * bv
                na = na + av * av
                nb = nb + bv * bv
            p = na * nb
            denom = jnp.maximum(p * _newton_rsqrt(p), jnp.float32(1e-8))
            out_v[pl.ds(g * 16, 16)] = num / denom
            return carry

        lax.fori_loop(0, _GROUPS, body, 0)

        pltpu.sync_copy(out_v, out_hbm.at[pl.ds(wid * _BPW, _BPW)])

    return k


_cosine_gather_kernel = _make_kernel()


@jax.jit
def kernel(entity_emb1, entity_emb2, entities):
    idx = entities.astype(jnp.int32).reshape(BATCH // _CHUNK, _CHUNK)
    return _cosine_gather_kernel(entity_emb1, entity_emb2, idx)


# speedup vs baseline: 2.6738x; 2.6738x over previous
import functools

import jax
import jax.numpy as jnp
from jax import lax
from jax.experimental import pallas as pl
from jax.experimental.pallas import tpu as pltpu
from jax.experimental.pallas import tpu_sc as plsc

DIM = 16
BATCH = 16384
_SUB = 8

_NC = 2
_NS = 16
_NW = _NC * _NS
_BPW = BATCH // _NW
_HALF = _BPW // 2
_HGROUPS = _HALF // 16


def _newton_rsqrt(p):
    i = plsc.bitcast(p, jnp.int32)
    i = jnp.int32(0x5F3759DF) - (i >> 1)
    y = plsc.bitcast(i, jnp.float32)
    for _ in range(3):
        y = y * (jnp.float32(1.5) - jnp.float32(0.5) * p * y * y)
    return y


def _make_kernel():
    mesh = plsc.VectorSubcoreMesh(core_axis_name="c", subcore_axis_name="s")

    @functools.partial(
        pl.kernel,
        mesh=mesh,
        out_type=jax.ShapeDtypeStruct((BATCH,), jnp.float32),
        compiler_params=pltpu.CompilerParams(needs_layout_passes=False),
        scratch_types=[
            pltpu.VMEM((_BPW,), jnp.int32),
            pltpu.VMEM((_HALF // _SUB, _SUB, DIM), jnp.float32),
            pltpu.VMEM((_HALF // _SUB, _SUB, DIM), jnp.float32),
            pltpu.VMEM((_BPW,), jnp.float32),
            pltpu.SemaphoreType.DMA,
        ],
    )
    def k(e1_hbm, e2_hbm, idx_hbm, out_hbm, raw_v, a_v, b_v, out_v, sem):
        wid = lax.axis_index("s") * _NC + lax.axis_index("c")
        base = wid * _BPW

        pltpu.sync_copy(idx_hbm.at[pl.ds(base, _BPW)], raw_v)

        lane = lax.iota(jnp.int32, 16)

        def gather_half(h):
            def issue(g, carry):
                vec = raw_v[pl.ds(h * _HALF + g * 16, 16)]
                for j in range(16):
                    r = vec[j]
                    t = r >> 3
                    s = r & jnp.int32(_SUB - 1)
                    e = g * 16 + j
                    pltpu.async_copy(
                        e1_hbm.at[t, s], a_v.at[e >> 3, e & (_SUB - 1)], sem)
                    pltpu.async_copy(
                        e2_hbm.at[t, s], b_v.at[e >> 3, e & (_SUB - 1)], sem)
                return carry

            lax.fori_loop(0, _HGROUPS, issue, 0)
            pltpu.make_async_copy(
                e1_hbm.at[pl.ds(0, _HALF // _SUB)], a_v, sem).wait()
            pltpu.make_async_copy(
                e2_hbm.at[pl.ds(0, _HALF // _SUB)], b_v, sem).wait()

        def compute_half(h):
            def body(g, carry):
                le = g * 16 + lane
                num = jnp.zeros((16,), jnp.float32)
                na = jnp.zeros((16,), jnp.float32)
                nb = jnp.zeros((16,), jnp.float32)
                for d in range(DIM):
                    col = jnp.full((16,), d, jnp.int32)
                    av = plsc.load_gather(
                        a_v, [le >> 3, le & jnp.int32(_SUB - 1), col])
                    bv = plsc.load_gather(
                        b_v, [le >> 3, le & jnp.int32(_SUB - 1), col])
                    num = num + av * bv
                    na = na + av * av
                    nb = nb + bv * bv
                p = na * nb
                denom = jnp.maximum(p * _newton_rsqrt(p), jnp.float32(1e-8))
                out_v[pl.ds(h * _HALF + g * 16, 16)] = num / denom
                return carry

            lax.fori_loop(0, _HGROUPS, body, 0)

        for h in range(2):
            gather_half(h)
            compute_half(h)

        pltpu.sync_copy(out_v, out_hbm.at[pl.ds(base, _BPW)])

    return k


_cosine_gather_kernel = _make_kernel()


@jax.jit
def kernel(entity_emb1, entity_emb2, entities):
    idx = entities.astype(jnp.int32)
    e1 = entity_emb1.reshape(-1, _SUB, DIM)
    e2 = entity_emb2.reshape(-1, _SUB, DIM)
    return _cosine_gather_kernel(e1, e2, idx)
